# baseline (speedup 1.0000x reference)
import jax
import jax.numpy as jnp
from jax.experimental import pallas as pl
from jax.experimental.pallas import tpu as pltpu

B, S, H, R = 4, 2048, 1024, 4
TILE = 512
GPAD = 128


def _fused_kernel(x_ref, wm_ref, bm_ref, gcol_ref, gb8_ref, be8_ref,
                  wg_ref, we_ref, wo_ref, bo_ref, out_ref,
                  wide_ref, wgs_ref, vec_ref, gv_ref):
    i = pl.program_id(0)

    @pl.when(i == 0)
    def _fold():
        wo = wo_ref[...]
        gcol = gcol_ref[...]
        gb8 = gb8_ref[...]
        for r in range(R):
            f = jnp.dot(we_ref[r], wo, preferred_element_type=jnp.float32)
            wide_ref[:, r * H:(r + 1) * H] = (gcol * f).astype(jnp.bfloat16)
            t = jnp.dot(gb8, we_ref[r],
                        preferred_element_type=jnp.float32) + be8_ref[r]
            s = jnp.dot(t.astype(jnp.bfloat16), wo,
                        preferred_element_type=jnp.float32)
            vec_ref[pl.ds(r, 1), :] = s[1:2, :]
            vec_ref[pl.ds(R + r, 1), :] = s[0:1, :]
        wgs_ref[...] = (gcol * wg_ref[...]).astype(jnp.bfloat16)
        gv_ref[...] = jnp.dot(gb8, wg_ref[...],
                              preferred_element_type=jnp.float32)

    x = x_ref[...]
    v = x + jnp.dot(x.astype(jnp.bfloat16), wm_ref[...],
                    preferred_element_type=jnp.float32) + bm_ref[...]
    vb = v.astype(jnp.bfloat16)

    mu = jnp.mean(v, axis=-1, keepdims=True)
    var = jnp.mean((v - mu) ** 2, axis=-1, keepdims=True)
    inv = jax.lax.rsqrt(var + 1e-5)
    muinv = mu * inv

    lg = jnp.dot(vb, wgs_ref[...],
                 preferred_element_type=jnp.float32)
    logits = inv * lg[:, :R] - muinv * gv_ref[0:1, :R] + gv_ref[1:2, :R]
    m = jnp.max(logits, axis=-1, keepdims=True)
    e = jnp.exp(logits - m)
    p = e / jnp.sum(e, axis=-1, keepdims=True)
    q = p * inv

    pq = jnp.concatenate([p, -muinv * p], axis=1)
    acc = jnp.dot(pq.astype(jnp.bfloat16),
                  vec_ref[...].astype(jnp.bfloat16),
                  preferred_element_type=jnp.float32) + bo_ref[...]
    for r in range(R):
        acc += q[:, r:r + 1] * jnp.dot(
            vb, wide_ref[:, r * H:(r + 1) * H],
            preferred_element_type=jnp.float32)
    out_ref[...] = acc


def kernel(x, Wm, bm, gamma, beta, Wg, We, be, Wo, bo):
    xf = x.reshape(B * S, H)
    n_tiles = (B * S) // TILE
    wg_pad = jnp.zeros((H, GPAD), jnp.float32).at[:, :R].set(Wg)
    gb8 = jnp.zeros((8, H), jnp.float32).at[0].set(gamma).at[1].set(beta)
    be8 = jnp.zeros((R, 8, H), jnp.float32).at[:, 1, :].set(be)
    full = lambda *shape: pl.BlockSpec(shape, lambda i: (0,) * len(shape))
    out = pl.pallas_call(
        _fused_kernel,
        grid=(n_tiles,),
        in_specs=[
            pl.BlockSpec((TILE, H), lambda i: (i, 0)),
            full(H, H),
            full(1, H),
            full(H, 1),
            full(8, H),
            full(R, 8, H),
            full(H, GPAD),
            full(R, H, H),
            full(H, H),
            full(1, H),
        ],
        out_specs=pl.BlockSpec((TILE, H), lambda i: (i, 0)),
        out_shape=jax.ShapeDtypeStruct((B * S, H), jnp.float32),
        scratch_shapes=[
            pltpu.VMEM((H, R * H), jnp.bfloat16),
            pltpu.VMEM((H, GPAD), jnp.bfloat16),
            pltpu.VMEM((2 * R, H), jnp.float32),
            pltpu.VMEM((8, GPAD), jnp.float32),
        ],
    )(xf, Wm.astype(jnp.bfloat16), bm.reshape(1, H), gamma.reshape(H, 1),
      gb8, be8, wg_pad, We.astype(jnp.bfloat16), Wo.astype(jnp.bfloat16),
      bo.reshape(1, H))
    return out.reshape(B, S, H)


# speedup vs baseline: 1.0214x; 1.0214x over previous
import jax
import jax.numpy as jnp
from jax.experimental import pallas as pl
from jax.experimental.pallas import tpu as pltpu

B, S, H, R = 4, 2048, 1024, 4
TILE = 512
GPAD = 128
KW = R * H + GPAD


def _fused_kernel(x_ref, wm_ref, bm_ref, gcol_ref, gb8_ref, be8_ref,
                  wg_ref, we_ref, wo_ref, bo8_ref, out_ref,
                  wm1_ref, wide_ref, gv_ref):
    i = pl.program_id(0)

    @pl.when(i == 0)
    def _fold():
        wo = wo_ref[...]
        gcol = gcol_ref[...]
        gb8 = gb8_ref[...]
        wgg = gcol * wg_ref[...]
        wm_b = wm_ref[...]
        wm1_ref[:, :H] = wm_b
        wm1_ref[:, H:] = (wgg + jnp.dot(
            wm_b, wgg.astype(jnp.bfloat16),
            preferred_element_type=jnp.float32)).astype(jnp.bfloat16)
        gv_ref[...] = jnp.dot(gb8, wg_ref[...],
                              preferred_element_type=jnp.float32)
        for r0 in range(0, R, 2):
            fa = jnp.dot(we_ref[r0], wo, preferred_element_type=jnp.float32)
            fb = jnp.dot(we_ref[r0 + 1], wo,
                         preferred_element_type=jnp.float32)
            wide_ref[pl.ds(r0 * H, H), :] = (gcol * fa).astype(jnp.bfloat16)
            wide_ref[pl.ds((r0 + 1) * H, H), :] = (
                gcol * fb).astype(jnp.bfloat16)
        wide_ref[pl.ds(R * H, GPAD), :] = jnp.zeros((GPAD, H), jnp.bfloat16)
        for r in range(R):
            t = jnp.dot(gb8, we_ref[r],
                        preferred_element_type=jnp.float32) + be8_ref[r]
            s = jnp.dot(t.astype(jnp.bfloat16), wo,
                        preferred_element_type=jnp.float32)
            wide_ref[pl.ds(R * H + r, 1), :] = s[1:2, :].astype(jnp.bfloat16)
            wide_ref[pl.ds(R * H + R + r, 1), :] = (
                s[0:1, :].astype(jnp.bfloat16))
        wide_ref[pl.ds(R * H + 2 * R, 8), :] = bo8_ref[...].astype(
            jnp.bfloat16)

    x = x_ref[...]
    pre = jnp.dot(x.astype(jnp.bfloat16), wm1_ref[...],
                  preferred_element_type=jnp.float32)
    v = x + pre[:, :H] + bm_ref[...]

    mu = jnp.mean(v, axis=-1, keepdims=True)
    var = jnp.mean((v - mu) ** 2, axis=-1, keepdims=True)
    inv = jax.lax.rsqrt(var + 1e-5)
    muinv = mu * inv

    logits = (inv * (pre[:, H:H + R] + gv_ref[2:3, :R])
              - muinv * gv_ref[0:1, :R] + gv_ref[1:2, :R])
    m = jnp.max(logits, axis=-1, keepdims=True)
    e = jnp.exp(logits - m)
    p = e / jnp.sum(e, axis=-1, keepdims=True)
    q = p * inv

    tail = jnp.concatenate(
        [p, -muinv * p, jnp.ones((TILE, 1), jnp.float32),
         jnp.zeros((TILE, GPAD - 2 * R - 1), jnp.float32)], axis=1)
    a = jnp.concatenate(
        [q[:, r:r + 1] * v for r in range(R)] + [tail],
        axis=1).astype(jnp.bfloat16)
    out_ref[...] = jnp.dot(a, wide_ref[...],
                           preferred_element_type=jnp.float32)


def kernel(x, Wm, bm, gamma, beta, Wg, We, be, Wo, bo):
    xf = x.reshape(B * S, H)
    n_tiles = (B * S) // TILE
    wg_pad = jnp.zeros((H, GPAD), jnp.float32).at[:, :R].set(Wg)
    gb8 = (jnp.zeros((8, H), jnp.float32)
           .at[0].set(gamma).at[1].set(beta).at[2].set(bm * gamma))
    be8 = jnp.zeros((R, 8, H), jnp.float32).at[:, 1, :].set(be)
    bo8 = jnp.zeros((8, H), jnp.float32).at[0].set(bo)
    full = lambda *shape: pl.BlockSpec(shape, lambda i: (0,) * len(shape))
    out = pl.pallas_call(
        _fused_kernel,
        grid=(n_tiles,),
        in_specs=[
            pl.BlockSpec((TILE, H), lambda i: (i, 0)),
            full(H, H),
            full(1, H),
            full(H, 1),
            full(8, H),
            full(R, 8, H),
            full(H, GPAD),
            full(R, H, H),
            full(H, H),
            full(8, H),
        ],
        out_specs=pl.BlockSpec((TILE, H), lambda i: (i, 0)),
        out_shape=jax.ShapeDtypeStruct((B * S, H), jnp.float32),
        scratch_shapes=[
            pltpu.VMEM((H, H + GPAD), jnp.bfloat16),
            pltpu.VMEM((KW, H), jnp.bfloat16),
            pltpu.VMEM((8, GPAD), jnp.float32),
        ],
    )(xf, Wm.astype(jnp.bfloat16), bm.reshape(1, H), gamma.reshape(H, 1),
      gb8, be8, wg_pad, We.astype(jnp.bfloat16), Wo.astype(jnp.bfloat16),
      bo8)
    return out.reshape(B, S, H)


# speedup vs baseline: 1.0241x; 1.0026x over previous
import jax
import jax.numpy as jnp
from jax.experimental import pallas as pl
from jax.experimental.pallas import tpu as pltpu

B, S, H, R = 4, 2048, 1024, 4
TILE = 1024
NSUB = 2
ST = TILE // NSUB
GPAD = 128
KW = R * H + GPAD


def _fused_kernel(x_ref, wm_ref, bm_ref, gcol_ref, gb8_ref, be8_ref,
                  wg_ref, we_ref, wo_ref, bo8_ref, out_ref,
                  wm1_ref, wide_ref, gv_ref):
    i = pl.program_id(0)

    @pl.when(i == 0)
    def _fold():
        wo = wo_ref[...]
        gcol = gcol_ref[...]
        gb8 = gb8_ref[...]
        wgg = gcol * wg_ref[...]
        wm_b = wm_ref[...]
        wm1_ref[:, :H] = wm_b
        wm1_ref[:, H:] = (wgg + jnp.dot(
            wm_b, wgg.astype(jnp.bfloat16),
            preferred_element_type=jnp.float32)).astype(jnp.bfloat16)
        gv_ref[...] = jnp.dot(gb8, wg_ref[...],
                              preferred_element_type=jnp.float32)
        for r0 in range(0, R, 2):
            fa = jnp.dot(we_ref[r0], wo, preferred_element_type=jnp.float32)
            fb = jnp.dot(we_ref[r0 + 1], wo,
                         preferred_element_type=jnp.float32)
            wide_ref[pl.ds(r0 * H, H), :] = (gcol * fa).astype(jnp.bfloat16)
            wide_ref[pl.ds((r0 + 1) * H, H), :] = (
                gcol * fb).astype(jnp.bfloat16)
        wide_ref[pl.ds(R * H, GPAD), :] = jnp.zeros((GPAD, H), jnp.bfloat16)
        for r in range(R):
            t = jnp.dot(gb8, we_ref[r],
                        preferred_element_type=jnp.float32) + be8_ref[r]
            s = jnp.dot(t.astype(jnp.bfloat16), wo,
                        preferred_element_type=jnp.float32)
            wide_ref[pl.ds(R * H + r, 1), :] = s[1:2, :].astype(jnp.bfloat16)
            wide_ref[pl.ds(R * H + R + r, 1), :] = (
                s[0:1, :].astype(jnp.bfloat16))
        wide_ref[pl.ds(R * H + 2 * R, 8), :] = bo8_ref[...].astype(
            jnp.bfloat16)

    for j in range(NSUB):
        x = x_ref[pl.ds(j * ST, ST), :]
        pre = jnp.dot(x.astype(jnp.bfloat16), wm1_ref[...],
                      preferred_element_type=jnp.float32)
        v = x + pre[:, :H] + bm_ref[...]

        mu = jnp.mean(v, axis=-1, keepdims=True)
        var = jnp.mean((v - mu) ** 2, axis=-1, keepdims=True)
        inv = jax.lax.rsqrt(var + 1e-5)
        muinv = mu * inv

        logits = (inv * (pre[:, H:H + R] + gv_ref[2:3, :R])
                  - muinv * gv_ref[0:1, :R] + gv_ref[1:2, :R])
        m = jnp.max(logits, axis=-1, keepdims=True)
        e = jnp.exp(logits - m)
        p = e / jnp.sum(e, axis=-1, keepdims=True)
        q = p * inv

        tail = jnp.concatenate(
            [p, -muinv * p, jnp.ones((ST, 1), jnp.float32),
             jnp.zeros((ST, GPAD - 2 * R - 1), jnp.float32)], axis=1)
        a = jnp.concatenate(
            [q[:, r:r + 1] * v for r in range(R)] + [tail],
            axis=1).astype(jnp.bfloat16)
        out_ref[pl.ds(j * ST, ST), :] = jnp.dot(
            a, wide_ref[...], preferred_element_type=jnp.float32)


def kernel(x, Wm, bm, gamma, beta, Wg, We, be, Wo, bo):
    xf = x.reshape(B * S, H)
    n_tiles = (B * S) // TILE
    wg_pad = jnp.zeros((H, GPAD), jnp.float32).at[:, :R].set(Wg)
    gb8 = (jnp.zeros((8, H), jnp.float32)
           .at[0].set(gamma).at[1].set(beta).at[2].set(bm * gamma))
    be8 = jnp.zeros((R, 8, H), jnp.float32).at[:, 1, :].set(be)
    bo8 = jnp.zeros((8, H), jnp.float32).at[0].set(bo)
    full = lambda *shape: pl.BlockSpec(shape, lambda i: (0,) * len(shape))
    out = pl.pallas_call(
        _fused_kernel,
        grid=(n_tiles,),
        in_specs=[
            pl.BlockSpec((TILE, H), lambda i: (i, 0)),
            full(H, H),
            full(1, H),
            full(H, 1),
            full(8, H),
            full(R, 8, H),
            full(H, GPAD),
            full(R, H, H),
            full(H, H),
            full(8, H),
        ],
        out_specs=pl.BlockSpec((TILE, H), lambda i: (i, 0)),
        out_shape=jax.ShapeDtypeStruct((B * S, H), jnp.float32),
        scratch_shapes=[
            pltpu.VMEM((H, H + GPAD), jnp.bfloat16),
            pltpu.VMEM((KW, H), jnp.bfloat16),
            pltpu.VMEM((8, GPAD), jnp.float32),
        ],
    )(xf, Wm.astype(jnp.bfloat16), bm.reshape(1, H), gamma.reshape(H, 1),
      gb8, be8, wg_pad, We.astype(jnp.bfloat16), Wo.astype(jnp.bfloat16),
      bo8)
    return out.reshape(B, S, H)
